# initial kernel scaffold (unmeasured)
import jax
import jax.numpy as jnp
from jax import lax
from jax.experimental import pallas as pl
from jax.experimental.pallas import tpu as pltpu

N_DEV = 4
M = 4096
N_TOT = 8192
HALF = N_TOT // 2
CHUNK = M // N_DEV
TILE = 512


def _ar_body(p_ref, o_ref,
             recv_r, recv_l,
             rs_send_r, rs_recv_r, rs_send_l, rs_recv_l,
             ag_send_r, ag_recv_r, ag_send_l, ag_recv_l,
             tile_a, tile_b, sem_a, sem_b):
    my = lax.axis_index("i")
    right = lax.rem(my + 1, N_DEV)
    left = lax.rem(my + N_DEV - 1, N_DEV)

    barrier = pltpu.get_barrier_semaphore()
    for nbr in (left, right):
        pl.semaphore_signal(barrier, inc=1, device_id=(nbr,),
                            device_id_type=pl.DeviceIdType.MESH)
    pl.semaphore_wait(barrier, 2)

    def rows(c):
        return pl.ds(c * CHUNK, CHUNK)

    def tiled_add(chunk_idx, col0, recv_slot):
        base = chunk_idx * CHUNK
        for t in range(CHUNK // TILE):
            r = base + t * TILE
            cp_a = pltpu.make_async_copy(
                p_ref.at[pl.ds(r, TILE), pl.ds(col0, HALF)], tile_a, sem_a)
            cp_b = pltpu.make_async_copy(
                recv_slot.at[pl.ds(t * TILE, TILE), :], tile_b, sem_b)
            cp_a.start()
            cp_b.start()
            cp_a.wait()
            cp_b.wait()
            tile_a[...] = tile_a[...] + tile_b[...]
            cp_o = pltpu.make_async_copy(
                tile_a, o_ref.at[pl.ds(r, TILE), pl.ds(col0, HALF)], sem_a)
            cp_o.start()
            cp_o.wait()

    for s in range(N_DEV - 1):
        cr_s = lax.rem(my - s + N_DEV, N_DEV)
        cl_s = lax.rem(my + s, N_DEV)
        src = p_ref if s == 0 else o_ref
        rd_r = pltpu.make_async_remote_copy(
            src_ref=src.at[rows(cr_s), pl.ds(0, HALF)],
            dst_ref=recv_r.at[s],
            send_sem=rs_send_r.at[s], recv_sem=rs_recv_r.at[s],
            device_id=(right,), device_id_type=pl.DeviceIdType.MESH)
        rd_l = pltpu.make_async_remote_copy(
            src_ref=src.at[rows(cl_s), pl.ds(HALF, HALF)],
            dst_ref=recv_l.at[s],
            send_sem=rs_send_l.at[s], recv_sem=rs_recv_l.at[s],
            device_id=(left,), device_id_type=pl.DeviceIdType.MESH)
        rd_r.start()
        rd_l.start()
        rd_r.wait()
        rd_l.wait()
        cr_a = lax.rem(my - s - 1 + N_DEV, N_DEV)
        cl_a = lax.rem(my + s + 1, N_DEV)
        tiled_add(cr_a, 0, recv_r.at[s])
        tiled_add(cl_a, HALF, recv_l.at[s])

    for s in range(N_DEV - 1):
        cr = lax.rem(my + 1 - s + N_DEV, N_DEV)
        cl = lax.rem(my - 1 + s + N_DEV, N_DEV)
        rd_r = pltpu.make_async_remote_copy(
            src_ref=o_ref.at[rows(cr), pl.ds(0, HALF)],
            dst_ref=o_ref.at[rows(cr), pl.ds(0, HALF)],
            send_sem=ag_send_r.at[s], recv_sem=ag_recv_r.at[s],
            device_id=(right,), device_id_type=pl.DeviceIdType.MESH)
        rd_l = pltpu.make_async_remote_copy(
            src_ref=o_ref.at[rows(cl), pl.ds(HALF, HALF)],
            dst_ref=o_ref.at[rows(cl), pl.ds(HALF, HALF)],
            send_sem=ag_send_l.at[s], recv_sem=ag_recv_l.at[s],
            device_id=(left,), device_id_type=pl.DeviceIdType.MESH)
        rd_r.start()
        rd_l.start()
        rd_r.wait()
        rd_l.wait()


def _all_reduce(partial):
    n_steps = N_DEV - 1
    return pl.pallas_call(
        _ar_body,
        out_shape=jax.ShapeDtypeStruct((M, N_TOT), jnp.float32),
        in_specs=[pl.BlockSpec(memory_space=pl.ANY)],
        out_specs=pl.BlockSpec(memory_space=pl.ANY),
        scratch_shapes=[
            pltpu.HBM((n_steps, CHUNK, HALF), jnp.float32),
            pltpu.HBM((n_steps, CHUNK, HALF), jnp.float32),
            pltpu.SemaphoreType.DMA((n_steps,)),
            pltpu.SemaphoreType.DMA((n_steps,)),
            pltpu.SemaphoreType.DMA((n_steps,)),
            pltpu.SemaphoreType.DMA((n_steps,)),
            pltpu.SemaphoreType.DMA((n_steps,)),
            pltpu.SemaphoreType.DMA((n_steps,)),
            pltpu.SemaphoreType.DMA((n_steps,)),
            pltpu.SemaphoreType.DMA((n_steps,)),
            pltpu.VMEM((TILE, HALF), jnp.float32),
            pltpu.VMEM((TILE, HALF), jnp.float32),
            pltpu.SemaphoreType.DMA,
            pltpu.SemaphoreType.DMA,
        ],
        compiler_params=pltpu.CompilerParams(collective_id=0),
    )(partial)


def kernel(x, w_mat):
    partial = jnp.dot(x, w_mat, preferred_element_type=jnp.float32)
    y = _all_reduce(partial)
    amax = jnp.max(jnp.abs(y))
    scale = amax / 127.0
    q = jnp.clip(jnp.round(y / scale), -127.0, 127.0)
    return (q * scale).astype(jnp.float32)


# baseline (device time: 1426118 ns/iter reference)
import jax
import jax.numpy as jnp
from jax import lax
from jax.experimental import pallas as pl
from jax.experimental.pallas import tpu as pltpu

N_DEV = 4
M = 4096
N_TOT = 8192
HALF = N_TOT // 2
CHUNK = M // N_DEV
TILE = 512


def _ar_body(p_ref, o_ref, recv_r, recv_l,
             rs_send_r, rs_recv_r, rs_send_l, rs_recv_l,
             ag_send_r, ag_recv_r, ag_send_l, ag_recv_l,
             tile_a, tile_b, sem_a, sem_b):
    my = lax.axis_index("i")
    right = lax.rem(my + 1, N_DEV)
    left = lax.rem(my + N_DEV - 1, N_DEV)

    barrier = pltpu.get_barrier_semaphore()
    for nbr in (left, right):
        pl.semaphore_signal(barrier, inc=1, device_id=(nbr,),
                            device_id_type=pl.DeviceIdType.MESH)
    pl.semaphore_wait(barrier, 2)

    def rows(c):
        return pl.ds(c * CHUNK, CHUNK)

    def tiled_add(chunk_idx, col0, recv_slot):
        base = chunk_idx * CHUNK
        for t in range(CHUNK // TILE):
            r = base + t * TILE
            cp_a = pltpu.make_async_copy(
                p_ref.at[pl.ds(r, TILE), pl.ds(col0, HALF)], tile_a, sem_a)
            cp_b = pltpu.make_async_copy(
                recv_slot.at[pl.ds(t * TILE, TILE), :], tile_b, sem_b)
            cp_a.start()
            cp_b.start()
            cp_a.wait()
            cp_b.wait()
            tile_a[...] = tile_a[...] + tile_b[...]
            cp_o = pltpu.make_async_copy(
                tile_a, o_ref.at[pl.ds(r, TILE), pl.ds(col0, HALF)], sem_a)
            cp_o.start()
            cp_o.wait()

    for s in range(N_DEV - 1):
        cr_s = lax.rem(my - s + N_DEV, N_DEV)
        cl_s = lax.rem(my + s, N_DEV)
        src = p_ref if s == 0 else o_ref
        rd_r = pltpu.make_async_remote_copy(
            src_ref=src.at[rows(cr_s), pl.ds(0, HALF)],
            dst_ref=recv_r.at[s],
            send_sem=rs_send_r.at[s], recv_sem=rs_recv_r.at[s],
            device_id=(right,), device_id_type=pl.DeviceIdType.MESH)
        rd_l = pltpu.make_async_remote_copy(
            src_ref=src.at[rows(cl_s), pl.ds(HALF, HALF)],
            dst_ref=recv_l.at[s],
            send_sem=rs_send_l.at[s], recv_sem=rs_recv_l.at[s],
            device_id=(left,), device_id_type=pl.DeviceIdType.MESH)
        rd_r.start()
        rd_l.start()
        rd_r.wait()
        rd_l.wait()
        cr_a = lax.rem(my - s - 1 + N_DEV, N_DEV)
        cl_a = lax.rem(my + s + 1, N_DEV)
        tiled_add(cr_a, 0, recv_r.at[s])
        tiled_add(cl_a, HALF, recv_l.at[s])

    for s in range(N_DEV - 1):
        cr = lax.rem(my + 1 - s + N_DEV, N_DEV)
        cl = lax.rem(my - 1 + s + N_DEV, N_DEV)
        rd_r = pltpu.make_async_remote_copy(
            src_ref=o_ref.at[rows(cr), pl.ds(0, HALF)],
            dst_ref=o_ref.at[rows(cr), pl.ds(0, HALF)],
            send_sem=ag_send_r.at[s], recv_sem=ag_recv_r.at[s],
            device_id=(right,), device_id_type=pl.DeviceIdType.MESH)
        rd_l = pltpu.make_async_remote_copy(
            src_ref=o_ref.at[rows(cl), pl.ds(HALF, HALF)],
            dst_ref=o_ref.at[rows(cl), pl.ds(HALF, HALF)],
            send_sem=ag_send_l.at[s], recv_sem=ag_recv_l.at[s],
            device_id=(left,), device_id_type=pl.DeviceIdType.MESH)
        rd_r.start()
        rd_l.start()
        rd_r.wait()
        rd_l.wait()


def _all_reduce(partial):
    n_steps = N_DEV - 1
    y, _, _ = pl.pallas_call(
        _ar_body,
        out_shape=(
            jax.ShapeDtypeStruct((M, N_TOT), jnp.float32),
            jax.ShapeDtypeStruct((n_steps, CHUNK, HALF), jnp.float32),
            jax.ShapeDtypeStruct((n_steps, CHUNK, HALF), jnp.float32),
        ),
        in_specs=[pl.BlockSpec(memory_space=pl.ANY)],
        out_specs=(
            pl.BlockSpec(memory_space=pl.ANY),
            pl.BlockSpec(memory_space=pl.ANY),
            pl.BlockSpec(memory_space=pl.ANY),
        ),
        scratch_shapes=[
            pltpu.SemaphoreType.DMA((n_steps,)),
            pltpu.SemaphoreType.DMA((n_steps,)),
            pltpu.SemaphoreType.DMA((n_steps,)),
            pltpu.SemaphoreType.DMA((n_steps,)),
            pltpu.SemaphoreType.DMA((n_steps,)),
            pltpu.SemaphoreType.DMA((n_steps,)),
            pltpu.SemaphoreType.DMA((n_steps,)),
            pltpu.SemaphoreType.DMA((n_steps,)),
            pltpu.VMEM((TILE, HALF), jnp.float32),
            pltpu.VMEM((TILE, HALF), jnp.float32),
            pltpu.SemaphoreType.DMA,
            pltpu.SemaphoreType.DMA,
        ],
        compiler_params=pltpu.CompilerParams(collective_id=0),
    )(partial)
    return y


def kernel(x, w_mat):
    partial = jnp.dot(x, w_mat, preferred_element_type=jnp.float32)
    y = _all_reduce(partial)
    amax = jnp.max(jnp.abs(y))
    scale = amax / 127.0
    q = jnp.clip(jnp.round(y / scale), -127.0, 127.0)
    return (q * scale).astype(jnp.float32)


# device time: 1324110 ns/iter; 1.0770x vs baseline; 1.0770x over previous
import jax
import jax.numpy as jnp
from jax import lax
from jax.experimental import pallas as pl
from jax.experimental.pallas import tpu as pltpu

N_DEV = 4
M = 4096
N_TOT = 8192
HALF = N_TOT // 2
CHUNK = M // N_DEV
TILE = 512
N_SUB = CHUNK // TILE


def _ar_body(p_ref, o_ref, recv_r, recv_l,
             rs_send_r, rs_recv_r, rs_send_l, rs_recv_l,
             ag_send_r, ag_recv_r, ag_send_l, ag_recv_l,
             tile_a, tile_b, sem_a, sem_b):
    my = lax.axis_index("i")
    right = lax.rem(my + 1, N_DEV)
    left = lax.rem(my + N_DEV - 1, N_DEV)

    barrier = pltpu.get_barrier_semaphore()
    for nbr in (left, right):
        pl.semaphore_signal(barrier, inc=1, device_id=(nbr,),
                            device_id_type=pl.DeviceIdType.MESH)
    pl.semaphore_wait(barrier, 2)

    def rows(c):
        return pl.ds(c * CHUNK, CHUNK)

    def add_sub(chunk_idx, b, col0, recv_buf, s):
        r = chunk_idx * CHUNK + b * TILE
        cp_a = pltpu.make_async_copy(
            p_ref.at[pl.ds(r, TILE), pl.ds(col0, HALF)], tile_a, sem_a)
        cp_b = pltpu.make_async_copy(
            recv_buf.at[s, pl.ds(b * TILE, TILE), :], tile_b, sem_b)
        cp_a.start()
        cp_b.start()
        cp_a.wait()
        cp_b.wait()
        tile_a[...] = tile_a[...] + tile_b[...]
        cp_o = pltpu.make_async_copy(
            tile_a, o_ref.at[pl.ds(r, TILE), pl.ds(col0, HALF)], sem_a)
        cp_o.start()
        cp_o.wait()

    def mk_rs_r(s, b):
        c = lax.rem(my - s + N_DEV, N_DEV)
        src = p_ref if s == 0 else o_ref
        return pltpu.make_async_remote_copy(
            src_ref=src.at[pl.ds(c * CHUNK + b * TILE, TILE), pl.ds(0, HALF)],
            dst_ref=recv_r.at[s, pl.ds(b * TILE, TILE), :],
            send_sem=rs_send_r.at[s, b], recv_sem=rs_recv_r.at[s, b],
            device_id=(right,), device_id_type=pl.DeviceIdType.MESH)

    def mk_rs_l(s, b):
        c = lax.rem(my + s, N_DEV)
        src = p_ref if s == 0 else o_ref
        return pltpu.make_async_remote_copy(
            src_ref=src.at[pl.ds(c * CHUNK + b * TILE, TILE),
                           pl.ds(HALF, HALF)],
            dst_ref=recv_l.at[s, pl.ds(b * TILE, TILE), :],
            send_sem=rs_send_l.at[s, b], recv_sem=rs_recv_l.at[s, b],
            device_id=(left,), device_id_type=pl.DeviceIdType.MESH)

    n_steps = N_DEV - 1
    rs_r = [[None] * N_SUB for _ in range(n_steps)]
    rs_l = [[None] * N_SUB for _ in range(n_steps)]
    for b in range(N_SUB):
        rs_r[0][b] = mk_rs_r(0, b)
        rs_r[0][b].start()
        rs_l[0][b] = mk_rs_l(0, b)
        rs_l[0][b].start()
    for s in range(n_steps):
        cr_a = lax.rem(my - s - 1 + N_DEV, N_DEV)
        cl_a = lax.rem(my + s + 1, N_DEV)
        for b in range(N_SUB):
            rs_r[s][b].wait()
            add_sub(cr_a, b, 0, recv_r, s)
            if s + 1 < n_steps:
                rs_r[s + 1][b] = mk_rs_r(s + 1, b)
                rs_r[s + 1][b].start()
            rs_l[s][b].wait()
            add_sub(cl_a, b, HALF, recv_l, s)
            if s + 1 < n_steps:
                rs_l[s + 1][b] = mk_rs_l(s + 1, b)
                rs_l[s + 1][b].start()

    for s in range(N_DEV - 1):
        cr = lax.rem(my + 1 - s + N_DEV, N_DEV)
        cl = lax.rem(my - 1 + s + N_DEV, N_DEV)
        rd_r = pltpu.make_async_remote_copy(
            src_ref=o_ref.at[rows(cr), pl.ds(0, HALF)],
            dst_ref=o_ref.at[rows(cr), pl.ds(0, HALF)],
            send_sem=ag_send_r.at[s], recv_sem=ag_recv_r.at[s],
            device_id=(right,), device_id_type=pl.DeviceIdType.MESH)
        rd_l = pltpu.make_async_remote_copy(
            src_ref=o_ref.at[rows(cl), pl.ds(HALF, HALF)],
            dst_ref=o_ref.at[rows(cl), pl.ds(HALF, HALF)],
            send_sem=ag_send_l.at[s], recv_sem=ag_recv_l.at[s],
            device_id=(left,), device_id_type=pl.DeviceIdType.MESH)
        rd_r.start()
        rd_l.start()
        rd_r.wait()
        rd_l.wait()


def _all_reduce(partial):
    n_steps = N_DEV - 1
    y, _, _ = pl.pallas_call(
        _ar_body,
        out_shape=(
            jax.ShapeDtypeStruct((M, N_TOT), jnp.float32),
            jax.ShapeDtypeStruct((n_steps, CHUNK, HALF), jnp.float32),
            jax.ShapeDtypeStruct((n_steps, CHUNK, HALF), jnp.float32),
        ),
        in_specs=[pl.BlockSpec(memory_space=pl.ANY)],
        out_specs=(
            pl.BlockSpec(memory_space=pl.ANY),
            pl.BlockSpec(memory_space=pl.ANY),
            pl.BlockSpec(memory_space=pl.ANY),
        ),
        scratch_shapes=[
            pltpu.SemaphoreType.DMA((n_steps, N_SUB)),
            pltpu.SemaphoreType.DMA((n_steps, N_SUB)),
            pltpu.SemaphoreType.DMA((n_steps, N_SUB)),
            pltpu.SemaphoreType.DMA((n_steps, N_SUB)),
            pltpu.SemaphoreType.DMA((n_steps,)),
            pltpu.SemaphoreType.DMA((n_steps,)),
            pltpu.SemaphoreType.DMA((n_steps,)),
            pltpu.SemaphoreType.DMA((n_steps,)),
            pltpu.VMEM((TILE, HALF), jnp.float32),
            pltpu.VMEM((TILE, HALF), jnp.float32),
            pltpu.SemaphoreType.DMA,
            pltpu.SemaphoreType.DMA,
        ],
        compiler_params=pltpu.CompilerParams(collective_id=0),
    )(partial)
    return y


def kernel(x, w_mat):
    partial = jnp.dot(x, w_mat, preferred_element_type=jnp.float32)
    y = _all_reduce(partial)
    amax = jnp.max(jnp.abs(y))
    scale = amax / 127.0
    q = jnp.clip(jnp.round(y / scale), -127.0, 127.0)
    return (q * scale).astype(jnp.float32)


# device time: 940410 ns/iter; 1.5165x vs baseline; 1.4080x over previous
import jax
import jax.numpy as jnp
from jax import lax
from jax.experimental import pallas as pl
from jax.experimental.pallas import tpu as pltpu

N_DEV = 4
M = 4096
N_TOT = 8192
HALF = N_TOT // 2
CHUNK = M // N_DEV
TILE = 512
N_SUB = CHUNK // TILE
N_STEPS = N_DEV - 1


def _ar_body(p_ref, o_ref, recv_r, recv_l, q_ref,
             rs_send_r, rs_recv_r, rs_send_l, rs_recv_l,
             ag_send_r, ag_recv_r, ag_send_l, ag_recv_l,
             amax_ssem, amax_rsem,
             tile_a, tile_b, tile_q, amax_src, amax_recv,
             sem_a, sem_b):
    my = lax.axis_index("i")
    right = lax.rem(my + 1, N_DEV)
    left = lax.rem(my + N_DEV - 1, N_DEV)
    opp = lax.rem(my + 2, N_DEV)

    barrier = pltpu.get_barrier_semaphore()
    for nbr in (left, right):
        pl.semaphore_signal(barrier, inc=1, device_id=(nbr,),
                            device_id_type=pl.DeviceIdType.MESH)
    pl.semaphore_wait(barrier, 2)

    def add_sub(chunk_idx, b, col0, recv_buf, s, want_amax):
        r = chunk_idx * CHUNK + b * TILE
        cp_a = pltpu.make_async_copy(
            p_ref.at[pl.ds(r, TILE), pl.ds(col0, HALF)], tile_a, sem_a)
        cp_b = pltpu.make_async_copy(
            recv_buf.at[s, pl.ds(b * TILE, TILE), :], tile_b, sem_b)
        cp_a.start()
        cp_b.start()
        cp_a.wait()
        cp_b.wait()
        tile_a[...] = tile_a[...] + tile_b[...]
        amax = jnp.max(jnp.abs(tile_a[...])) if want_amax else None
        cp_o = pltpu.make_async_copy(
            tile_a, o_ref.at[pl.ds(r, TILE), pl.ds(col0, HALF)], sem_a)
        cp_o.start()
        cp_o.wait()
        return amax

    def mk_rs_r(s, b):
        c = lax.rem(my - s + N_DEV, N_DEV)
        src = p_ref if s == 0 else o_ref
        return pltpu.make_async_remote_copy(
            src_ref=src.at[pl.ds(c * CHUNK + b * TILE, TILE), pl.ds(0, HALF)],
            dst_ref=recv_r.at[s, pl.ds(b * TILE, TILE), :],
            send_sem=rs_send_r.at[s, b], recv_sem=rs_recv_r.at[s, b],
            device_id=(right,), device_id_type=pl.DeviceIdType.MESH)

    def mk_rs_l(s, b):
        c = lax.rem(my + s, N_DEV)
        src = p_ref if s == 0 else o_ref
        return pltpu.make_async_remote_copy(
            src_ref=src.at[pl.ds(c * CHUNK + b * TILE, TILE),
                           pl.ds(HALF, HALF)],
            dst_ref=recv_l.at[s, pl.ds(b * TILE, TILE), :],
            send_sem=rs_send_l.at[s, b], recv_sem=rs_recv_l.at[s, b],
            device_id=(left,), device_id_type=pl.DeviceIdType.MESH)

    local_amax = jnp.float32(0.0)
    rs_r = [[None] * N_SUB for _ in range(N_STEPS)]
    rs_l = [[None] * N_SUB for _ in range(N_STEPS)]
    for b in range(N_SUB):
        rs_r[0][b] = mk_rs_r(0, b)
        rs_r[0][b].start()
        rs_l[0][b] = mk_rs_l(0, b)
        rs_l[0][b].start()
    for s in range(N_STEPS):
        cr_a = lax.rem(my - s - 1 + N_DEV, N_DEV)
        cl_a = lax.rem(my + s + 1, N_DEV)
        last = s == N_STEPS - 1
        for b in range(N_SUB):
            rs_r[s][b].wait()
            m1 = add_sub(cr_a, b, 0, recv_r, s, last)
            if not last:
                rs_r[s + 1][b] = mk_rs_r(s + 1, b)
                rs_r[s + 1][b].start()
            rs_l[s][b].wait()
            m2 = add_sub(cl_a, b, HALF, recv_l, s, last)
            if not last:
                rs_l[s + 1][b] = mk_rs_l(s + 1, b)
                rs_l[s + 1][b].start()
            if last:
                local_amax = jnp.maximum(local_amax, jnp.maximum(m1, m2))

    amax_src[...] = jnp.full((8, 128), local_amax, jnp.float32)
    ex = []
    for k, tgt in ((0, right), (1, left), (2, opp)):
        rd = pltpu.make_async_remote_copy(
            src_ref=amax_src, dst_ref=amax_recv.at[k],
            send_sem=amax_ssem.at[k], recv_sem=amax_rsem.at[k],
            device_id=(tgt,), device_id_type=pl.DeviceIdType.MESH)
        rd.start()
        ex.append(rd)
    for rd in ex:
        rd.wait()
    g_amax = jnp.maximum(local_amax, jnp.max(amax_recv[...]))
    scale = g_amax / 127.0
    inv_scale = 127.0 / g_amax

    def quant_own(chunk_idx, b, col0):
        r = chunk_idx * CHUNK + b * TILE
        cp_i = pltpu.make_async_copy(
            o_ref.at[pl.ds(r, TILE), pl.ds(col0, HALF)], tile_a, sem_a)
        cp_i.start()
        cp_i.wait()
        qf = jnp.clip(jnp.round(tile_a[...] * inv_scale), -127.0, 127.0)
        tile_q[...] = qf.astype(jnp.int8)
        tile_a[...] = qf * scale
        cp_q = pltpu.make_async_copy(
            tile_q, q_ref.at[pl.ds(r, TILE), pl.ds(col0, HALF)], sem_b)
        cp_o = pltpu.make_async_copy(
            tile_a, o_ref.at[pl.ds(r, TILE), pl.ds(col0, HALF)], sem_a)
        cp_q.start()
        cp_o.start()
        cp_q.wait()
        cp_o.wait()

    def dequant_sub(chunk_idx, b, col0):
        r = chunk_idx * CHUNK + b * TILE
        cp_i = pltpu.make_async_copy(
            q_ref.at[pl.ds(r, TILE), pl.ds(col0, HALF)], tile_q, sem_b)
        cp_i.start()
        cp_i.wait()
        tile_a[...] = tile_q[...].astype(jnp.float32) * scale
        cp_o = pltpu.make_async_copy(
            tile_a, o_ref.at[pl.ds(r, TILE), pl.ds(col0, HALF)], sem_a)
        cp_o.start()
        cp_o.wait()

    own_r = lax.rem(my + 1, N_DEV)
    own_l = lax.rem(my - 1 + N_DEV, N_DEV)
    for b in range(N_SUB):
        quant_own(own_r, b, 0)
        quant_own(own_l, b, HALF)

    def mk_ag_r(s):
        c = lax.rem(my + 1 - s + N_DEV, N_DEV)
        reg = q_ref.at[pl.ds(c * CHUNK, CHUNK), pl.ds(0, HALF)]
        return pltpu.make_async_remote_copy(
            src_ref=reg, dst_ref=reg,
            send_sem=ag_send_r.at[s], recv_sem=ag_recv_r.at[s],
            device_id=(right,), device_id_type=pl.DeviceIdType.MESH)

    def mk_ag_l(s):
        c = lax.rem(my - 1 + s + N_DEV, N_DEV)
        reg = q_ref.at[pl.ds(c * CHUNK, CHUNK), pl.ds(HALF, HALF)]
        return pltpu.make_async_remote_copy(
            src_ref=reg, dst_ref=reg,
            send_sem=ag_send_l.at[s], recv_sem=ag_recv_l.at[s],
            device_id=(left,), device_id_type=pl.DeviceIdType.MESH)

    ag_r = mk_ag_r(0)
    ag_l = mk_ag_l(0)
    ag_r.start()
    ag_l.start()
    for s in range(N_STEPS):
        ag_r.wait()
        ag_l.wait()
        if s + 1 < N_STEPS:
            ag_r = mk_ag_r(s + 1)
            ag_l = mk_ag_l(s + 1)
            ag_r.start()
            ag_l.start()
        cr = lax.rem(my - s + N_DEV, N_DEV)
        cl = lax.rem(my + s, N_DEV)
        for b in range(N_SUB):
            dequant_sub(cr, b, 0)
            dequant_sub(cl, b, HALF)


def _all_reduce_quant(partial):
    y, _, _, _ = pl.pallas_call(
        _ar_body,
        out_shape=(
            jax.ShapeDtypeStruct((M, N_TOT), jnp.float32),
            jax.ShapeDtypeStruct((N_STEPS, CHUNK, HALF), jnp.float32),
            jax.ShapeDtypeStruct((N_STEPS, CHUNK, HALF), jnp.float32),
            jax.ShapeDtypeStruct((M, N_TOT), jnp.int8),
        ),
        in_specs=[pl.BlockSpec(memory_space=pl.ANY)],
        out_specs=(
            pl.BlockSpec(memory_space=pl.ANY),
            pl.BlockSpec(memory_space=pl.ANY),
            pl.BlockSpec(memory_space=pl.ANY),
            pl.BlockSpec(memory_space=pl.ANY),
        ),
        scratch_shapes=[
            pltpu.SemaphoreType.DMA((N_STEPS, N_SUB)),
            pltpu.SemaphoreType.DMA((N_STEPS, N_SUB)),
            pltpu.SemaphoreType.DMA((N_STEPS, N_SUB)),
            pltpu.SemaphoreType.DMA((N_STEPS, N_SUB)),
            pltpu.SemaphoreType.DMA((N_STEPS,)),
            pltpu.SemaphoreType.DMA((N_STEPS,)),
            pltpu.SemaphoreType.DMA((N_STEPS,)),
            pltpu.SemaphoreType.DMA((N_STEPS,)),
            pltpu.SemaphoreType.DMA((3,)),
            pltpu.SemaphoreType.DMA((3,)),
            pltpu.VMEM((TILE, HALF), jnp.float32),
            pltpu.VMEM((TILE, HALF), jnp.float32),
            pltpu.VMEM((TILE, HALF), jnp.int8),
            pltpu.VMEM((8, 128), jnp.float32),
            pltpu.VMEM((3, 8, 128), jnp.float32),
            pltpu.SemaphoreType.DMA,
            pltpu.SemaphoreType.DMA,
        ],
        compiler_params=pltpu.CompilerParams(collective_id=0),
    )(partial)
    return y


def kernel(x, w_mat):
    partial = jnp.dot(x, w_mat, preferred_element_type=jnp.float32)
    return _all_reduce_quant(partial)


# device time: 676177 ns/iter; 2.1091x vs baseline; 1.3908x over previous
import jax
import jax.numpy as jnp
from jax import lax
from jax.experimental import pallas as pl
from jax.experimental.pallas import tpu as pltpu

N_DEV = 4
M = 4096
N_TOT = 8192
HALF = N_TOT // 2
CHUNK = M // N_DEV
TILE = 512
N_SUB = CHUNK // TILE
N_STEPS = N_DEV - 1


def _ar_body(p_ref, o_ref, recv_r, recv_l, q_ref, sbuf_r, sbuf_l,
             rs_send_r, rs_recv_r, rs_send_l, rs_recv_l,
             ag_send_r, ag_recv_r, ag_send_l, ag_recv_l,
             amax_ssem, amax_rsem,
             tile_a, tile_bf, tile_q, amax_src, amax_recv,
             sem_a, sem_b):
    my = lax.axis_index("i")
    right = lax.rem(my + 1, N_DEV)
    left = lax.rem(my + N_DEV - 1, N_DEV)
    opp = lax.rem(my + 2, N_DEV)

    barrier = pltpu.get_barrier_semaphore()
    for nbr in (left, right):
        pl.semaphore_signal(barrier, inc=1, device_id=(nbr,),
                            device_id_type=pl.DeviceIdType.MESH)
    pl.semaphore_wait(barrier, 2)

    def stage0(chunk_idx, b, col0, sbuf):
        r = chunk_idx * CHUNK + b * TILE
        cp_a = pltpu.make_async_copy(
            p_ref.at[pl.ds(r, TILE), pl.ds(col0, HALF)], tile_a, sem_a)
        cp_a.start()
        cp_a.wait()
        tile_bf[...] = tile_a[...].astype(jnp.bfloat16)
        cp_s = pltpu.make_async_copy(
            tile_bf, sbuf.at[0, pl.ds(b * TILE, TILE), :], sem_b)
        cp_s.start()
        cp_s.wait()

    def add_sub(chunk_idx, b, col0, recv_buf, sbuf, s, want_amax):
        r = chunk_idx * CHUNK + b * TILE
        cp_a = pltpu.make_async_copy(
            p_ref.at[pl.ds(r, TILE), pl.ds(col0, HALF)], tile_a, sem_a)
        cp_b = pltpu.make_async_copy(
            recv_buf.at[s, pl.ds(b * TILE, TILE), :], tile_bf, sem_b)
        cp_a.start()
        cp_b.start()
        cp_a.wait()
        cp_b.wait()
        tile_a[...] = tile_a[...] + tile_bf[...].astype(jnp.float32)
        if not want_amax:
            tile_bf[...] = tile_a[...].astype(jnp.bfloat16)
            cp_s = pltpu.make_async_copy(
                tile_bf, sbuf.at[s + 1, pl.ds(b * TILE, TILE), :], sem_b)
            cp_s.start()
            cp_s.wait()
            return None
        amax = jnp.max(jnp.abs(tile_a[...]))
        cp_o = pltpu.make_async_copy(
            tile_a, o_ref.at[pl.ds(r, TILE), pl.ds(col0, HALF)], sem_a)
        cp_o.start()
        cp_o.wait()
        return amax

    def mk_rs_r(s, b):
        return pltpu.make_async_remote_copy(
            src_ref=sbuf_r.at[s, pl.ds(b * TILE, TILE), :],
            dst_ref=recv_r.at[s, pl.ds(b * TILE, TILE), :],
            send_sem=rs_send_r.at[s, b], recv_sem=rs_recv_r.at[s, b],
            device_id=(right,), device_id_type=pl.DeviceIdType.MESH)

    def mk_rs_l(s, b):
        return pltpu.make_async_remote_copy(
            src_ref=sbuf_l.at[s, pl.ds(b * TILE, TILE), :],
            dst_ref=recv_l.at[s, pl.ds(b * TILE, TILE), :],
            send_sem=rs_send_l.at[s, b], recv_sem=rs_recv_l.at[s, b],
            device_id=(left,), device_id_type=pl.DeviceIdType.MESH)

    local_amax = jnp.float32(0.0)
    rs_r = [[None] * N_SUB for _ in range(N_STEPS)]
    rs_l = [[None] * N_SUB for _ in range(N_STEPS)]
    for b in range(N_SUB):
        stage0(my, b, 0, sbuf_r)
        rs_r[0][b] = mk_rs_r(0, b)
        rs_r[0][b].start()
        stage0(my, b, HALF, sbuf_l)
        rs_l[0][b] = mk_rs_l(0, b)
        rs_l[0][b].start()
    for s in range(N_STEPS):
        cr_a = lax.rem(my - s - 1 + N_DEV, N_DEV)
        cl_a = lax.rem(my + s + 1, N_DEV)
        last = s == N_STEPS - 1
        for b in range(N_SUB):
            rs_r[s][b].wait()
            m1 = add_sub(cr_a, b, 0, recv_r, sbuf_r, s, last)
            if not last:
                rs_r[s + 1][b] = mk_rs_r(s + 1, b)
                rs_r[s + 1][b].start()
            rs_l[s][b].wait()
            m2 = add_sub(cl_a, b, HALF, recv_l, sbuf_l, s, last)
            if not last:
                rs_l[s + 1][b] = mk_rs_l(s + 1, b)
                rs_l[s + 1][b].start()
            if last:
                local_amax = jnp.maximum(local_amax, jnp.maximum(m1, m2))

    amax_src[...] = jnp.full((8, 128), local_amax, jnp.float32)
    ex = []
    for k, tgt in ((0, right), (1, left), (2, opp)):
        rd = pltpu.make_async_remote_copy(
            src_ref=amax_src, dst_ref=amax_recv.at[k],
            send_sem=amax_ssem.at[k], recv_sem=amax_rsem.at[k],
            device_id=(tgt,), device_id_type=pl.DeviceIdType.MESH)
        rd.start()
        ex.append(rd)
    for rd in ex:
        rd.wait()
    g_amax = jnp.maximum(local_amax, jnp.max(amax_recv[...]))
    scale = g_amax / 127.0
    inv_scale = 127.0 / g_amax

    def quant_own(chunk_idx, b, col0):
        r = chunk_idx * CHUNK + b * TILE
        cp_i = pltpu.make_async_copy(
            o_ref.at[pl.ds(r, TILE), pl.ds(col0, HALF)], tile_a, sem_a)
        cp_i.start()
        cp_i.wait()
        qf = jnp.clip(jnp.round(tile_a[...] * inv_scale), -127.0, 127.0)
        tile_q[...] = qf.astype(jnp.int8)
        tile_a[...] = qf * scale
        cp_q = pltpu.make_async_copy(
            tile_q, q_ref.at[pl.ds(r, TILE), pl.ds(col0, HALF)], sem_b)
        cp_o = pltpu.make_async_copy(
            tile_a, o_ref.at[pl.ds(r, TILE), pl.ds(col0, HALF)], sem_a)
        cp_q.start()
        cp_o.start()
        cp_q.wait()
        cp_o.wait()

    def dequant_sub(chunk_idx, b, col0):
        r = chunk_idx * CHUNK + b * TILE
        cp_i = pltpu.make_async_copy(
            q_ref.at[pl.ds(r, TILE), pl.ds(col0, HALF)], tile_q, sem_b)
        cp_i.start()
        cp_i.wait()
        tile_a[...] = tile_q[...].astype(jnp.float32) * scale
        cp_o = pltpu.make_async_copy(
            tile_a, o_ref.at[pl.ds(r, TILE), pl.ds(col0, HALF)], sem_a)
        cp_o.start()
        cp_o.wait()

    own_r = lax.rem(my + 1, N_DEV)
    own_l = lax.rem(my - 1 + N_DEV, N_DEV)
    for b in range(N_SUB):
        quant_own(own_r, b, 0)
        quant_own(own_l, b, HALF)

    def mk_ag_r(s):
        c = lax.rem(my + 1 - s + N_DEV, N_DEV)
        reg = q_ref.at[pl.ds(c * CHUNK, CHUNK), pl.ds(0, HALF)]
        return pltpu.make_async_remote_copy(
            src_ref=reg, dst_ref=reg,
            send_sem=ag_send_r.at[s], recv_sem=ag_recv_r.at[s],
            device_id=(right,), device_id_type=pl.DeviceIdType.MESH)

    def mk_ag_l(s):
        c = lax.rem(my - 1 + s + N_DEV, N_DEV)
        reg = q_ref.at[pl.ds(c * CHUNK, CHUNK), pl.ds(HALF, HALF)]
        return pltpu.make_async_remote_copy(
            src_ref=reg, dst_ref=reg,
            send_sem=ag_send_l.at[s], recv_sem=ag_recv_l.at[s],
            device_id=(left,), device_id_type=pl.DeviceIdType.MESH)

    ag_r = mk_ag_r(0)
    ag_l = mk_ag_l(0)
    ag_r.start()
    ag_l.start()
    for s in range(N_STEPS):
        ag_r.wait()
        ag_l.wait()
        if s + 1 < N_STEPS:
            ag_r = mk_ag_r(s + 1)
            ag_l = mk_ag_l(s + 1)
            ag_r.start()
            ag_l.start()
        cr = lax.rem(my - s + N_DEV, N_DEV)
        cl = lax.rem(my + s, N_DEV)
        for b in range(N_SUB):
            dequant_sub(cr, b, 0)
            dequant_sub(cl, b, HALF)


def _all_reduce_quant(partial):
    y, _, _, _, _, _ = pl.pallas_call(
        _ar_body,
        out_shape=(
            jax.ShapeDtypeStruct((M, N_TOT), jnp.float32),
            jax.ShapeDtypeStruct((N_STEPS, CHUNK, HALF), jnp.bfloat16),
            jax.ShapeDtypeStruct((N_STEPS, CHUNK, HALF), jnp.bfloat16),
            jax.ShapeDtypeStruct((M, N_TOT), jnp.int8),
            jax.ShapeDtypeStruct((N_STEPS, CHUNK, HALF), jnp.bfloat16),
            jax.ShapeDtypeStruct((N_STEPS, CHUNK, HALF), jnp.bfloat16),
        ),
        in_specs=[pl.BlockSpec(memory_space=pl.ANY)],
        out_specs=tuple(pl.BlockSpec(memory_space=pl.ANY) for _ in range(6)),
        scratch_shapes=[
            pltpu.SemaphoreType.DMA((N_STEPS, N_SUB)),
            pltpu.SemaphoreType.DMA((N_STEPS, N_SUB)),
            pltpu.SemaphoreType.DMA((N_STEPS, N_SUB)),
            pltpu.SemaphoreType.DMA((N_STEPS, N_SUB)),
            pltpu.SemaphoreType.DMA((N_STEPS,)),
            pltpu.SemaphoreType.DMA((N_STEPS,)),
            pltpu.SemaphoreType.DMA((N_STEPS,)),
            pltpu.SemaphoreType.DMA((N_STEPS,)),
            pltpu.SemaphoreType.DMA((3,)),
            pltpu.SemaphoreType.DMA((3,)),
            pltpu.VMEM((TILE, HALF), jnp.float32),
            pltpu.VMEM((TILE, HALF), jnp.bfloat16),
            pltpu.VMEM((TILE, HALF), jnp.int8),
            pltpu.VMEM((8, 128), jnp.float32),
            pltpu.VMEM((3, 8, 128), jnp.float32),
            pltpu.SemaphoreType.DMA,
            pltpu.SemaphoreType.DMA,
        ],
        compiler_params=pltpu.CompilerParams(collective_id=0),
    )(partial)
    return y


def kernel(x, w_mat):
    partial = jnp.dot(x, w_mat, preferred_element_type=jnp.float32)
    return _all_reduce_quant(partial)


# device time: 633434 ns/iter; 2.2514x vs baseline; 1.0675x over previous
import jax
import jax.numpy as jnp
from jax import lax
from jax.experimental import pallas as pl
from jax.experimental.pallas import tpu as pltpu

N_DEV = 4
M = 4096
N_TOT = 8192
HALF = N_TOT // 2
CHUNK = M // N_DEV
TILE = 512
N_SUB = CHUNK // TILE
N_STEPS = N_DEV - 1


def _ar_body(p_ref, o_ref, recv_r, recv_l, q_ref, sbuf_r, sbuf_l,
             rs_send_r, rs_recv_r, rs_send_l, rs_recv_l,
             ag_send_r, ag_recv_r, ag_send_l, ag_recv_l,
             amax_ssem, amax_rsem,
             tile_a, tile_bf, tile_q, tiles_keep, amax_src, amax_recv,
             sem_a, sem_b):
    my = lax.axis_index("i")
    right = lax.rem(my + 1, N_DEV)
    left = lax.rem(my + N_DEV - 1, N_DEV)
    opp = lax.rem(my + 2, N_DEV)

    barrier = pltpu.get_barrier_semaphore()
    for nbr in (left, right):
        pl.semaphore_signal(barrier, inc=1, device_id=(nbr,),
                            device_id_type=pl.DeviceIdType.MESH)
    pl.semaphore_wait(barrier, 2)

    def stage0(chunk_idx, b, col0, sbuf):
        r = chunk_idx * CHUNK + b * TILE
        cp_a = pltpu.make_async_copy(
            p_ref.at[pl.ds(r, TILE), pl.ds(col0, HALF)], tile_a, sem_a)
        cp_a.start()
        cp_a.wait()
        tile_bf[...] = tile_a[...].astype(jnp.bfloat16)
        cp_s = pltpu.make_async_copy(
            tile_bf, sbuf.at[0, pl.ds(b * TILE, TILE), :], sem_b)
        cp_s.start()
        cp_s.wait()

    def add_sub(chunk_idx, b, col0, recv_buf, sbuf, s, keep_k=None):
        r = chunk_idx * CHUNK + b * TILE
        cp_a = pltpu.make_async_copy(
            p_ref.at[pl.ds(r, TILE), pl.ds(col0, HALF)], tile_a, sem_a)
        cp_b = pltpu.make_async_copy(
            recv_buf.at[s, pl.ds(b * TILE, TILE), :], tile_bf, sem_b)
        cp_a.start()
        cp_b.start()
        cp_a.wait()
        cp_b.wait()
        if keep_k is None:
            tile_a[...] = tile_a[...] + tile_bf[...].astype(jnp.float32)
            tile_bf[...] = tile_a[...].astype(jnp.bfloat16)
            cp_s = pltpu.make_async_copy(
                tile_bf, sbuf.at[s + 1, pl.ds(b * TILE, TILE), :], sem_b)
            cp_s.start()
            cp_s.wait()
            return None
        tiles_keep[keep_k, :, :] = tile_a[...] + tile_bf[...].astype(jnp.float32)
        return jnp.max(jnp.abs(tiles_keep[keep_k, :, :]))

    def mk_rs_r(s, b):
        return pltpu.make_async_remote_copy(
            src_ref=sbuf_r.at[s, pl.ds(b * TILE, TILE), :],
            dst_ref=recv_r.at[s, pl.ds(b * TILE, TILE), :],
            send_sem=rs_send_r.at[s, b], recv_sem=rs_recv_r.at[s, b],
            device_id=(right,), device_id_type=pl.DeviceIdType.MESH)

    def mk_rs_l(s, b):
        return pltpu.make_async_remote_copy(
            src_ref=sbuf_l.at[s, pl.ds(b * TILE, TILE), :],
            dst_ref=recv_l.at[s, pl.ds(b * TILE, TILE), :],
            send_sem=rs_send_l.at[s, b], recv_sem=rs_recv_l.at[s, b],
            device_id=(left,), device_id_type=pl.DeviceIdType.MESH)

    local_amax = jnp.float32(0.0)
    rs_r = [[None] * N_SUB for _ in range(N_STEPS)]
    rs_l = [[None] * N_SUB for _ in range(N_STEPS)]
    for b in range(N_SUB):
        stage0(my, b, 0, sbuf_r)
        rs_r[0][b] = mk_rs_r(0, b)
        rs_r[0][b].start()
        stage0(my, b, HALF, sbuf_l)
        rs_l[0][b] = mk_rs_l(0, b)
        rs_l[0][b].start()
    for s in range(N_STEPS):
        cr_a = lax.rem(my - s - 1 + N_DEV, N_DEV)
        cl_a = lax.rem(my + s + 1, N_DEV)
        last = s == N_STEPS - 1
        for b in range(N_SUB):
            rs_r[s][b].wait()
            m1 = add_sub(cr_a, b, 0, recv_r, sbuf_r, s,
                         keep_k=b if last else None)
            if not last:
                rs_r[s + 1][b] = mk_rs_r(s + 1, b)
                rs_r[s + 1][b].start()
            rs_l[s][b].wait()
            m2 = add_sub(cl_a, b, HALF, recv_l, sbuf_l, s,
                         keep_k=N_SUB + b if last else None)
            if not last:
                rs_l[s + 1][b] = mk_rs_l(s + 1, b)
                rs_l[s + 1][b].start()
            if last:
                local_amax = jnp.maximum(local_amax, jnp.maximum(m1, m2))

    amax_src[...] = jnp.full((8, 128), local_amax, jnp.float32)
    ex = []
    for k, tgt in ((0, right), (1, left), (2, opp)):
        rd = pltpu.make_async_remote_copy(
            src_ref=amax_src, dst_ref=amax_recv.at[k],
            send_sem=amax_ssem.at[k], recv_sem=amax_rsem.at[k],
            device_id=(tgt,), device_id_type=pl.DeviceIdType.MESH)
        rd.start()
        ex.append(rd)
    for rd in ex:
        rd.wait()
    g_amax = jnp.maximum(local_amax, jnp.max(amax_recv[...]))
    scale = g_amax / 127.0
    inv_scale = 127.0 / g_amax

    own_r = lax.rem(my + 1, N_DEV)
    own_l = lax.rem(my - 1 + N_DEV, N_DEV)

    def quant_own(chunk_idx, b, col0, keep_k):
        r = chunk_idx * CHUNK + b * TILE
        qf = jnp.clip(jnp.round(tiles_keep[keep_k, :, :] * inv_scale),
                      -127.0, 127.0)
        tile_q[...] = qf.astype(jnp.int8)
        cp_q = pltpu.make_async_copy(
            tile_q, q_ref.at[pl.ds(r, TILE), pl.ds(col0, HALF)], sem_b)
        cp_q.start()
        cp_q.wait()

    def dequant_sub(chunk_idx, b, col0):
        r = chunk_idx * CHUNK + b * TILE
        cp_i = pltpu.make_async_copy(
            q_ref.at[pl.ds(r, TILE), pl.ds(col0, HALF)], tile_q, sem_b)
        cp_i.start()
        cp_i.wait()
        tile_a[...] = tile_q[...].astype(jnp.float32) * scale
        cp_o = pltpu.make_async_copy(
            tile_a, o_ref.at[pl.ds(r, TILE), pl.ds(col0, HALF)], sem_a)
        cp_o.start()
        cp_o.wait()

    for b in range(N_SUB):
        quant_own(own_r, b, 0, b)
        quant_own(own_l, b, HALF, N_SUB + b)

    def mk_ag_r(s, b):
        c = lax.rem(my + 1 - s + N_DEV, N_DEV)
        reg = q_ref.at[pl.ds(c * CHUNK + b * TILE, TILE), pl.ds(0, HALF)]
        return pltpu.make_async_remote_copy(
            src_ref=reg, dst_ref=reg,
            send_sem=ag_send_r.at[s, b], recv_sem=ag_recv_r.at[s, b],
            device_id=(right,), device_id_type=pl.DeviceIdType.MESH)

    def mk_ag_l(s, b):
        c = lax.rem(my - 1 + s + N_DEV, N_DEV)
        reg = q_ref.at[pl.ds(c * CHUNK + b * TILE, TILE), pl.ds(HALF, HALF)]
        return pltpu.make_async_remote_copy(
            src_ref=reg, dst_ref=reg,
            send_sem=ag_send_l.at[s, b], recv_sem=ag_recv_l.at[s, b],
            device_id=(left,), device_id_type=pl.DeviceIdType.MESH)

    ag_r = [[None] * N_SUB for _ in range(N_STEPS)]
    ag_l = [[None] * N_SUB for _ in range(N_STEPS)]
    for b in range(N_SUB):
        ag_r[0][b] = mk_ag_r(0, b)
        ag_r[0][b].start()
        ag_l[0][b] = mk_ag_l(0, b)
        ag_l[0][b].start()
    for b in range(N_SUB):
        dequant_sub(own_r, b, 0)
        dequant_sub(own_l, b, HALF)
    for s in range(N_STEPS):
        cr = lax.rem(my - s + N_DEV, N_DEV)
        cl = lax.rem(my + s, N_DEV)
        for b in range(N_SUB):
            ag_r[s][b].wait()
            if s + 1 < N_STEPS:
                ag_r[s + 1][b] = mk_ag_r(s + 1, b)
                ag_r[s + 1][b].start()
            ag_l[s][b].wait()
            if s + 1 < N_STEPS:
                ag_l[s + 1][b] = mk_ag_l(s + 1, b)
                ag_l[s + 1][b].start()
            dequant_sub(cr, b, 0)
            dequant_sub(cl, b, HALF)


def _all_reduce_quant(partial):
    y, _, _, _, _, _ = pl.pallas_call(
        _ar_body,
        out_shape=(
            jax.ShapeDtypeStruct((M, N_TOT), jnp.float32),
            jax.ShapeDtypeStruct((N_STEPS, CHUNK, HALF), jnp.bfloat16),
            jax.ShapeDtypeStruct((N_STEPS, CHUNK, HALF), jnp.bfloat16),
            jax.ShapeDtypeStruct((M, N_TOT), jnp.int8),
            jax.ShapeDtypeStruct((N_STEPS, CHUNK, HALF), jnp.bfloat16),
            jax.ShapeDtypeStruct((N_STEPS, CHUNK, HALF), jnp.bfloat16),
        ),
        in_specs=[pl.BlockSpec(memory_space=pl.ANY)],
        out_specs=tuple(pl.BlockSpec(memory_space=pl.ANY) for _ in range(6)),
        scratch_shapes=[
            pltpu.SemaphoreType.DMA((N_STEPS, N_SUB)),
            pltpu.SemaphoreType.DMA((N_STEPS, N_SUB)),
            pltpu.SemaphoreType.DMA((N_STEPS, N_SUB)),
            pltpu.SemaphoreType.DMA((N_STEPS, N_SUB)),
            pltpu.SemaphoreType.DMA((N_STEPS, N_SUB)),
            pltpu.SemaphoreType.DMA((N_STEPS, N_SUB)),
            pltpu.SemaphoreType.DMA((N_STEPS, N_SUB)),
            pltpu.SemaphoreType.DMA((N_STEPS, N_SUB)),
            pltpu.SemaphoreType.DMA((3,)),
            pltpu.SemaphoreType.DMA((3,)),
            pltpu.VMEM((TILE, HALF), jnp.float32),
            pltpu.VMEM((TILE, HALF), jnp.bfloat16),
            pltpu.VMEM((TILE, HALF), jnp.int8),
            pltpu.VMEM((2 * N_SUB, TILE, HALF), jnp.float32),
            pltpu.VMEM((8, 128), jnp.float32),
            pltpu.VMEM((3, 8, 128), jnp.float32),
            pltpu.SemaphoreType.DMA,
            pltpu.SemaphoreType.DMA,
        ],
        compiler_params=pltpu.CompilerParams(
            collective_id=0, vmem_limit_bytes=100_663_296),
    )(partial)
    return y


def kernel(x, w_mat):
    partial = jnp.dot(x, w_mat, preferred_element_type=jnp.float32)
    return _all_reduce_quant(partial)


# device time: 633412 ns/iter; 2.2515x vs baseline; 1.0000x over previous
import jax
import jax.numpy as jnp
from jax import lax
from jax.experimental import pallas as pl
from jax.experimental.pallas import tpu as pltpu

N_DEV = 4
M = 4096
N_TOT = 8192
HALF = N_TOT // 2
CHUNK = M // N_DEV
TILE = 512
N_SUB = CHUNK // TILE
N_STEPS = N_DEV - 1
N_KEEP = 2 * N_SUB


def _ar_body(p_ref, o_ref, recv_r, recv_l, q_ref, sbuf_r, sbuf_l,
             rs_send_r, rs_recv_r, rs_send_l, rs_recv_l,
             ag_send_r, ag_recv_r, ag_send_l, ag_recv_l,
             amax_ssem, amax_rsem,
             tile_a, tile_bf, tile_q, tiles_keep, amax_src, amax_recv,
             sem_a, sem_b, sem_o):
    my = lax.axis_index("i")
    right = lax.rem(my + 1, N_DEV)
    left = lax.rem(my + N_DEV - 1, N_DEV)
    opp = lax.rem(my + 2, N_DEV)

    barrier = pltpu.get_barrier_semaphore()
    for nbr in (left, right):
        pl.semaphore_signal(barrier, inc=1, device_id=(nbr,),
                            device_id_type=pl.DeviceIdType.MESH)
    pl.semaphore_wait(barrier, 2)

    def stage0(chunk_idx, b, col0, sbuf):
        r = chunk_idx * CHUNK + b * TILE
        cp_a = pltpu.make_async_copy(
            p_ref.at[pl.ds(r, TILE), pl.ds(col0, HALF)], tile_a, sem_a)
        cp_a.start()
        cp_a.wait()
        tile_bf[...] = tile_a[...].astype(jnp.bfloat16)
        cp_s = pltpu.make_async_copy(
            tile_bf, sbuf.at[0, pl.ds(b * TILE, TILE), :], sem_b)
        cp_s.start()
        cp_s.wait()

    def add_sub(chunk_idx, b, col0, recv_buf, sbuf, s, keep_k=None):
        r = chunk_idx * CHUNK + b * TILE
        cp_a = pltpu.make_async_copy(
            p_ref.at[pl.ds(r, TILE), pl.ds(col0, HALF)], tile_a, sem_a)
        cp_b = pltpu.make_async_copy(
            recv_buf.at[s, pl.ds(b * TILE, TILE), :], tile_bf, sem_b)
        cp_a.start()
        cp_b.start()
        cp_a.wait()
        cp_b.wait()
        if keep_k is None:
            tile_a[...] = tile_a[...] + tile_bf[...].astype(jnp.float32)
            tile_bf[...] = tile_a[...].astype(jnp.bfloat16)
            cp_s = pltpu.make_async_copy(
                tile_bf, sbuf.at[s + 1, pl.ds(b * TILE, TILE), :], sem_b)
            cp_s.start()
            cp_s.wait()
            return None
        tiles_keep[keep_k, :, :] = tile_a[...] + tile_bf[...].astype(jnp.float32)
        return jnp.max(jnp.abs(tiles_keep[keep_k, :, :]))

    def mk_rs_r(s, b):
        return pltpu.make_async_remote_copy(
            src_ref=sbuf_r.at[s, pl.ds(b * TILE, TILE), :],
            dst_ref=recv_r.at[s, pl.ds(b * TILE, TILE), :],
            send_sem=rs_send_r.at[s, b], recv_sem=rs_recv_r.at[s, b],
            device_id=(right,), device_id_type=pl.DeviceIdType.MESH)

    def mk_rs_l(s, b):
        return pltpu.make_async_remote_copy(
            src_ref=sbuf_l.at[s, pl.ds(b * TILE, TILE), :],
            dst_ref=recv_l.at[s, pl.ds(b * TILE, TILE), :],
            send_sem=rs_send_l.at[s, b], recv_sem=rs_recv_l.at[s, b],
            device_id=(left,), device_id_type=pl.DeviceIdType.MESH)

    local_amax = jnp.float32(0.0)
    rs_r = [[None] * N_SUB for _ in range(N_STEPS)]
    rs_l = [[None] * N_SUB for _ in range(N_STEPS)]
    for b in range(N_SUB):
        stage0(my, b, 0, sbuf_r)
        rs_r[0][b] = mk_rs_r(0, b)
        rs_r[0][b].start()
        stage0(my, b, HALF, sbuf_l)
        rs_l[0][b] = mk_rs_l(0, b)
        rs_l[0][b].start()
    for s in range(N_STEPS):
        cr_a = lax.rem(my - s - 1 + N_DEV, N_DEV)
        cl_a = lax.rem(my + s + 1, N_DEV)
        last = s == N_STEPS - 1
        for b in range(N_SUB):
            rs_r[s][b].wait()
            m1 = add_sub(cr_a, b, 0, recv_r, sbuf_r, s,
                         keep_k=b if last else None)
            if not last:
                rs_r[s + 1][b] = mk_rs_r(s + 1, b)
                rs_r[s + 1][b].start()
            rs_l[s][b].wait()
            m2 = add_sub(cl_a, b, HALF, recv_l, sbuf_l, s,
                         keep_k=N_SUB + b if last else None)
            if not last:
                rs_l[s + 1][b] = mk_rs_l(s + 1, b)
                rs_l[s + 1][b].start()
            if last:
                local_amax = jnp.maximum(local_amax, jnp.maximum(m1, m2))

    amax_src[...] = jnp.full((8, 128), local_amax, jnp.float32)
    ex = []
    for k, tgt in ((0, right), (1, left), (2, opp)):
        rd = pltpu.make_async_remote_copy(
            src_ref=amax_src, dst_ref=amax_recv.at[k],
            send_sem=amax_ssem.at[k], recv_sem=amax_rsem.at[k],
            device_id=(tgt,), device_id_type=pl.DeviceIdType.MESH)
        rd.start()
        ex.append(rd)
    for rd in ex:
        rd.wait()
    g_amax = jnp.maximum(local_amax, jnp.max(amax_recv[...]))
    scale = g_amax / 127.0
    inv_scale = 127.0 / g_amax

    own_r = lax.rem(my + 1, N_DEV)
    own_l = lax.rem(my - 1 + N_DEV, N_DEV)

    def quant_own(chunk_idx, b, col0, keep_k):
        r = chunk_idx * CHUNK + b * TILE
        qf = jnp.clip(jnp.round(tiles_keep[keep_k, :, :] * inv_scale),
                      -127.0, 127.0)
        tile_q[...] = qf.astype(jnp.int8)
        cp_q = pltpu.make_async_copy(
            tile_q, q_ref.at[pl.ds(r, TILE), pl.ds(col0, HALF)], sem_b)
        cp_q.start()
        cp_q.wait()

    for b in range(N_SUB):
        quant_own(own_r, b, 0, b)
        quant_own(own_l, b, HALF, N_SUB + b)

    deq_state = {"i": 0, "pending": [None] * N_KEEP}

    def dequant_sub(chunk_idx, b, col0):
        r = chunk_idx * CHUNK + b * TILE
        k = deq_state["i"] % N_KEEP
        deq_state["i"] += 1
        cp_i = pltpu.make_async_copy(
            q_ref.at[pl.ds(r, TILE), pl.ds(col0, HALF)], tile_q, sem_b)
        cp_i.start()
        if deq_state["pending"][k] is not None:
            deq_state["pending"][k].wait()
        cp_i.wait()
        tiles_keep[k, :, :] = tile_q[...].astype(jnp.float32) * scale
        cp_o = pltpu.make_async_copy(
            tiles_keep.at[k], o_ref.at[pl.ds(r, TILE), pl.ds(col0, HALF)],
            sem_o.at[k])
        cp_o.start()
        deq_state["pending"][k] = cp_o

    def mk_ag_r(s, b):
        c = lax.rem(my + 1 - s + N_DEV, N_DEV)
        reg = q_ref.at[pl.ds(c * CHUNK + b * TILE, TILE), pl.ds(0, HALF)]
        return pltpu.make_async_remote_copy(
            src_ref=reg, dst_ref=reg,
            send_sem=ag_send_r.at[s, b], recv_sem=ag_recv_r.at[s, b],
            device_id=(right,), device_id_type=pl.DeviceIdType.MESH)

    def mk_ag_l(s, b):
        c = lax.rem(my - 1 + s + N_DEV, N_DEV)
        reg = q_ref.at[pl.ds(c * CHUNK + b * TILE, TILE), pl.ds(HALF, HALF)]
        return pltpu.make_async_remote_copy(
            src_ref=reg, dst_ref=reg,
            send_sem=ag_send_l.at[s, b], recv_sem=ag_recv_l.at[s, b],
            device_id=(left,), device_id_type=pl.DeviceIdType.MESH)

    ag_r = [[None] * N_SUB for _ in range(N_STEPS)]
    ag_l = [[None] * N_SUB for _ in range(N_STEPS)]
    for b in range(N_SUB):
        ag_r[0][b] = mk_ag_r(0, b)
        ag_r[0][b].start()
        ag_l[0][b] = mk_ag_l(0, b)
        ag_l[0][b].start()
    for b in range(N_SUB):
        dequant_sub(own_r, b, 0)
        dequant_sub(own_l, b, HALF)
    for s in range(N_STEPS):
        cr = lax.rem(my - s + N_DEV, N_DEV)
        cl = lax.rem(my + s, N_DEV)
        for b in range(N_SUB):
            ag_r[s][b].wait()
            if s + 1 < N_STEPS:
                ag_r[s + 1][b] = mk_ag_r(s + 1, b)
                ag_r[s + 1][b].start()
            ag_l[s][b].wait()
            if s + 1 < N_STEPS:
                ag_l[s + 1][b] = mk_ag_l(s + 1, b)
                ag_l[s + 1][b].start()
            dequant_sub(cr, b, 0)
            dequant_sub(cl, b, HALF)
    for cp in deq_state["pending"]:
        if cp is not None:
            cp.wait()


def _all_reduce_quant(partial):
    y, _, _, _, _, _ = pl.pallas_call(
        _ar_body,
        out_shape=(
            jax.ShapeDtypeStruct((M, N_TOT), jnp.float32),
            jax.ShapeDtypeStruct((N_STEPS, CHUNK, HALF), jnp.bfloat16),
            jax.ShapeDtypeStruct((N_STEPS, CHUNK, HALF), jnp.bfloat16),
            jax.ShapeDtypeStruct((M, N_TOT), jnp.int8),
            jax.ShapeDtypeStruct((N_STEPS, CHUNK, HALF), jnp.bfloat16),
            jax.ShapeDtypeStruct((N_STEPS, CHUNK, HALF), jnp.bfloat16),
        ),
        in_specs=[pl.BlockSpec(memory_space=pl.ANY)],
        out_specs=tuple(pl.BlockSpec(memory_space=pl.ANY) for _ in range(6)),
        scratch_shapes=[
            pltpu.SemaphoreType.DMA((N_STEPS, N_SUB)),
            pltpu.SemaphoreType.DMA((N_STEPS, N_SUB)),
            pltpu.SemaphoreType.DMA((N_STEPS, N_SUB)),
            pltpu.SemaphoreType.DMA((N_STEPS, N_SUB)),
            pltpu.SemaphoreType.DMA((N_STEPS, N_SUB)),
            pltpu.SemaphoreType.DMA((N_STEPS, N_SUB)),
            pltpu.SemaphoreType.DMA((N_STEPS, N_SUB)),
            pltpu.SemaphoreType.DMA((N_STEPS, N_SUB)),
            pltpu.SemaphoreType.DMA((3,)),
            pltpu.SemaphoreType.DMA((3,)),
            pltpu.VMEM((TILE, HALF), jnp.float32),
            pltpu.VMEM((TILE, HALF), jnp.bfloat16),
            pltpu.VMEM((TILE, HALF), jnp.int8),
            pltpu.VMEM((N_KEEP, TILE, HALF), jnp.float32),
            pltpu.VMEM((8, 128), jnp.float32),
            pltpu.VMEM((3, 8, 128), jnp.float32),
            pltpu.SemaphoreType.DMA,
            pltpu.SemaphoreType.DMA,
            pltpu.SemaphoreType.DMA((N_KEEP,)),
        ],
        compiler_params=pltpu.CompilerParams(
            collective_id=0, vmem_limit_bytes=100_663_296),
    )(partial)
    return y


def kernel(x, w_mat):
    partial = jnp.dot(x, w_mat, preferred_element_type=jnp.float32)
    return _all_reduce_quant(partial)
